# baseline (speedup 1.0000x reference)
import functools

import jax
import jax.numpy as jnp
from jax import lax
from jax.experimental import pallas as pl
from jax.experimental.pallas import tpu as pltpu
from jax.experimental.pallas import tpu_sc as plsc

VOCAB = 1000000
DIM = 64
BATCH = 16384
NUM_NEG = 10
NUM_ROWS = NUM_NEG + 2

NC = 2
NS = 16
NW = NC * NS
CHUNK = BATCH // NW


def _sc_gather(input_emb, output_emb, idx_flat):
    mesh = plsc.VectorSubcoreMesh(core_axis_name="c", subcore_axis_name="s")

    @functools.partial(
        pl.kernel,
        mesh=mesh,
        out_type=jax.ShapeDtypeStruct((NUM_ROWS * BATCH, DIM), jnp.float32),
        compiler_params=pltpu.CompilerParams(use_tc_tiling_on_sc=False),
        scratch_types=[
            pltpu.VMEM((CHUNK,), jnp.int32),
            pltpu.VMEM((CHUNK, DIM), jnp.float32),
            pltpu.SemaphoreType.DMA,
        ],
    )
    def gather_kernel(in_hbm, out_hbm, idx_hbm, g_hbm, idx_v, rows_v, sem):
        wid = lax.axis_index("s") * NC + lax.axis_index("c")
        base = wid * CHUNK
        for j in range(NUM_ROWS):
            table = in_hbm if j == 0 else out_hbm
            off = j * BATCH + base
            pltpu.sync_copy(idx_hbm.at[pl.ds(off, CHUNK)], idx_v)
            pltpu.async_copy(table.at[idx_v], rows_v, sem).wait()
            pltpu.sync_copy(rows_v, g_hbm.at[pl.ds(off, CHUNK)])

    return gather_kernel(input_emb, output_emb, idx_flat)


BB = 1024


def _loss_kernel(g_ref, o_ref):
    i = pl.program_id(0)
    g = g_ref[...]
    center = g[0]
    scores = jnp.sum(center[None, :, :] * g[1:], axis=-1)
    def softplus(x):
        return jnp.maximum(x, 0.0) + jnp.log1p(jnp.exp(-jnp.abs(x)))
    block = jnp.sum(softplus(-scores[0])) + jnp.sum(softplus(scores[1:]))

    @pl.when(i == 0)
    def _():
        o_ref[...] = jnp.zeros_like(o_ref)

    o_ref[...] += block


def _tc_loss(gathered):
    g3 = gathered.reshape(NUM_ROWS, BATCH, DIM)
    nb = BATCH // BB
    out = pl.pallas_call(
        _loss_kernel,
        grid=(nb,),
        in_specs=[pl.BlockSpec((NUM_ROWS, BB, DIM), lambda i: (0, i, 0))],
        out_specs=pl.BlockSpec((1, 1), lambda i: (0, 0)),
        out_shape=jax.ShapeDtypeStruct((1, 1), jnp.float32),
    )(g3)
    return out[0, 0] / BATCH


def kernel(center_nodes, context_nodes, negative_nodes, input_emb, output_emb):
    idx = jnp.concatenate(
        [
            center_nodes.astype(jnp.int32)[None, :],
            context_nodes.astype(jnp.int32)[None, :],
            negative_nodes.astype(jnp.int32).T,
        ],
        axis=0,
    ).reshape(-1)
    gathered = _sc_gather(input_emb, output_emb, idx)
    return _tc_loss(gathered)
